# baseline (speedup 1.0000x reference)
import functools

import jax
import jax.numpy as jnp
from jax import lax
from jax.experimental import pallas as pl
from jax.experimental.pallas import tpu as pltpu
from jax.experimental.pallas import tpu_sc as plsc


def _sc_gather(table, idx):
    B = idx.shape[0]
    V, D = table.shape
    info = plsc.get_sparse_core_info()
    nc, ns = info.num_cores, info.num_subcores
    nw = nc * ns
    b_per_w = B // nw
    mesh = plsc.VectorSubcoreMesh(core_axis_name="c", subcore_axis_name="s")

    @functools.partial(
        pl.kernel,
        mesh=mesh,
        compiler_params=pltpu.CompilerParams(use_tc_tiling_on_sc=False),
        out_type=jax.ShapeDtypeStruct((B, D), jnp.float32),
        scratch_types=[
            pltpu.VMEM((b_per_w,), jnp.int32),
            pltpu.VMEM((b_per_w, D), jnp.float32),
            pltpu.SemaphoreType.DMA,
        ],
    )
    def gather_kernel(table_hbm, idx_hbm, out_hbm, idx_v, rows_v, sem):
        wid = lax.axis_index("s") * nc + lax.axis_index("c")
        base = wid * b_per_w
        pltpu.sync_copy(idx_hbm.at[pl.ds(base, b_per_w)], idx_v)
        pltpu.async_copy(table_hbm.at[idx_v], rows_v, sem).wait()
        pltpu.sync_copy(rows_v, out_hbm.at[pl.ds(base, b_per_w)])

    return gather_kernel(table, idx)


def _mm_body(e_ref, w_ref, b_ref, o_ref):
    acc = lax.dot_general(
        e_ref[...], w_ref[...],
        dimension_numbers=(((1,), (1,)), ((), ())),
        preferred_element_type=jnp.float32,
    )
    o_ref[...] = acc + b_ref[...]


def _projection(embeds, linear_w, linear_b, tile_v=1024):
    B, D = embeds.shape
    V = linear_w.shape[0]
    grid = (pl.cdiv(V, tile_v),)
    return pl.pallas_call(
        _mm_body,
        grid=grid,
        in_specs=[
            pl.BlockSpec((B, D), lambda j: (0, 0)),
            pl.BlockSpec((tile_v, D), lambda j: (j, 0)),
            pl.BlockSpec((1, tile_v), lambda j: (0, j)),
        ],
        out_specs=pl.BlockSpec((B, tile_v), lambda j: (0, j)),
        out_shape=jax.ShapeDtypeStruct((B, V), jnp.float32),
    )(embeds, linear_w, linear_b.reshape(1, V))


def kernel(inputs, embedding_table, linear_w, linear_b):
    idx = inputs.astype(jnp.int32)
    embeds = _sc_gather(embedding_table, idx)
    return _projection(embeds, linear_w, linear_b)


# speedup vs baseline: 1.0242x; 1.0242x over previous
import functools

import jax
import jax.numpy as jnp
from jax import lax
from jax.experimental import pallas as pl
from jax.experimental.pallas import tpu as pltpu
from jax.experimental.pallas import tpu_sc as plsc

_TILE_V = 1024
_NBUF = 4


def _sc_gather(table, idx):
    B = idx.shape[0]
    V, D = table.shape
    info = plsc.get_sparse_core_info()
    nc, ns = info.num_cores, info.num_subcores
    nw = nc * ns
    b_per_w = B // nw
    mesh = plsc.VectorSubcoreMesh(core_axis_name="c", subcore_axis_name="s")

    @functools.partial(
        pl.kernel,
        mesh=mesh,
        compiler_params=pltpu.CompilerParams(use_tc_tiling_on_sc=False),
        out_type=jax.ShapeDtypeStruct((B, D), jnp.float32),
        scratch_types=[
            pltpu.VMEM((b_per_w,), jnp.int32),
            pltpu.VMEM((b_per_w, D), jnp.float32),
            pltpu.SemaphoreType.DMA,
        ],
    )
    def gather_kernel(table_hbm, idx_hbm, out_hbm, idx_v, rows_v, sem):
        wid = lax.axis_index("s") * nc + lax.axis_index("c")
        base = wid * b_per_w
        pltpu.sync_copy(idx_hbm.at[pl.ds(base, b_per_w)], idx_v)
        pltpu.async_copy(table_hbm.at[idx_v], rows_v, sem).wait()
        pltpu.sync_copy(rows_v, out_hbm.at[pl.ds(base, b_per_w)])

    return gather_kernel(table, idx)


def _make_main_body(n_full, tile_v, nbuf):
    def body(e_ref, w_ref, b_ref, o_hbm, acc_ref, sems):
        j = pl.program_id(0)
        slot = lax.rem(j, nbuf)

        @pl.when(j >= nbuf)
        def _():
            pltpu.make_async_copy(
                acc_ref.at[slot], o_hbm.at[:, pl.ds(0, tile_v)], sems.at[slot]
            ).wait()

        acc_ref[slot] = lax.dot_general(
            e_ref[...], w_ref[...],
            dimension_numbers=(((1,), (1,)), ((), ())),
            preferred_element_type=jnp.float32,
        ) + b_ref[...]
        pltpu.make_async_copy(
            acc_ref.at[slot],
            o_hbm.at[:, pl.ds(j * tile_v, tile_v)],
            sems.at[slot],
        ).start()

        @pl.when(j == n_full - 1)
        def _():
            for step in range(max(n_full - nbuf, 0), n_full):
                s = step % nbuf
                pltpu.make_async_copy(
                    acc_ref.at[s], o_hbm.at[:, pl.ds(0, tile_v)], sems.at[s]
                ).wait()

    return body


def _tail_body(o_in_ref, e_ref, w_ref, b_ref, o_ref):
    o_ref[...] = lax.dot_general(
        e_ref[...], w_ref[...],
        dimension_numbers=(((1,), (1,)), ((), ())),
        preferred_element_type=jnp.float32,
    ) + b_ref[...]


def _projection(embeds, linear_w, linear_b, tile_v=_TILE_V, nbuf=_NBUF):
    B, D = embeds.shape
    V = linear_w.shape[0]
    n_tiles = pl.cdiv(V, tile_v)
    n_full = V // tile_v
    bias2d = linear_b.reshape(1, V)

    main = pl.pallas_call(
        _make_main_body(n_full, tile_v, nbuf),
        grid=(n_full,),
        in_specs=[
            pl.BlockSpec((B, D), lambda j: (0, 0)),
            pl.BlockSpec((tile_v, D), lambda j: (j, 0)),
            pl.BlockSpec((1, tile_v), lambda j: (0, j)),
        ],
        out_specs=pl.BlockSpec(memory_space=pl.ANY),
        out_shape=jax.ShapeDtypeStruct((B, V), jnp.float32),
        scratch_shapes=[
            pltpu.VMEM((nbuf, B, tile_v), jnp.float32),
            pltpu.SemaphoreType.DMA((nbuf,)),
        ],
    )(embeds, linear_w, bias2d)

    if n_tiles == n_full:
        return main

    last = n_tiles - 1
    return pl.pallas_call(
        _tail_body,
        grid=(1,),
        in_specs=[
            pl.BlockSpec(memory_space=pl.ANY),
            pl.BlockSpec((B, D), lambda j: (0, 0)),
            pl.BlockSpec((tile_v, D), lambda j: (last, 0)),
            pl.BlockSpec((1, tile_v), lambda j: (0, last)),
        ],
        out_specs=pl.BlockSpec((B, tile_v), lambda j: (0, last)),
        out_shape=jax.ShapeDtypeStruct((B, V), jnp.float32),
        input_output_aliases={0: 0},
    )(main, embeds, linear_w, bias2d)


def kernel(inputs, embedding_table, linear_w, linear_b):
    idx = inputs.astype(jnp.int32)
    embeds = _sc_gather(embedding_table, idx)
    return _projection(embeds, linear_w, linear_b)


# speedup vs baseline: 1.1401x; 1.1131x over previous
import functools

import jax
import jax.numpy as jnp
from jax import lax
from jax.experimental import pallas as pl
from jax.experimental.pallas import tpu as pltpu
from jax.experimental.pallas import tpu_sc as plsc

_TILE_V = 1024
_NBUF = 4


def _sc_gather(table, idx):
    B = idx.shape[0]
    V, D = table.shape
    info = plsc.get_sparse_core_info()
    nc, ns = info.num_cores, info.num_subcores
    nw = nc * ns
    b_per_w = B // nw
    mesh = plsc.VectorSubcoreMesh(core_axis_name="c", subcore_axis_name="s")

    @functools.partial(
        pl.kernel,
        mesh=mesh,
        compiler_params=pltpu.CompilerParams(use_tc_tiling_on_sc=False),
        out_type=jax.ShapeDtypeStruct((B, D), jnp.float32),
        scratch_types=[
            pltpu.VMEM((b_per_w,), jnp.int32),
            pltpu.VMEM((b_per_w, D), jnp.float32),
            pltpu.SemaphoreType.DMA,
        ],
    )
    def gather_kernel(table_hbm, idx_hbm, out_hbm, idx_v, rows_v, sem):
        wid = lax.axis_index("s") * nc + lax.axis_index("c")
        base = wid * b_per_w
        pltpu.sync_copy(idx_hbm.at[pl.ds(base, b_per_w)], idx_v)
        pltpu.async_copy(table_hbm.at[idx_v], rows_v, sem).wait()
        pltpu.sync_copy(rows_v, out_hbm.at[pl.ds(base, b_per_w)])

    return gather_kernel(table, idx)


def _make_main_body(n_full, tile_v, nbuf):
    def body(e_ref, w_ref, b_ref, o_hbm, acc_ref, sems):
        j = pl.program_id(0)
        slot = lax.rem(j, nbuf)

        @pl.when(j >= nbuf)
        def _():
            pltpu.make_async_copy(
                acc_ref.at[slot], o_hbm.at[:, pl.ds(0, tile_v)], sems.at[slot]
            ).wait()

        acc_ref[slot] = lax.dot_general(
            e_ref[...], w_ref[...],
            dimension_numbers=(((1,), (1,)), ((), ())),
            preferred_element_type=jnp.float32,
        ) + b_ref[...]
        pltpu.make_async_copy(
            acc_ref.at[slot],
            o_hbm.at[:, pl.ds(j * tile_v, tile_v)],
            sems.at[slot],
        ).start()

        @pl.when(j == n_full - 1)
        def _():
            for step in range(max(n_full - nbuf, 0), n_full):
                s = step % nbuf
                pltpu.make_async_copy(
                    acc_ref.at[s], o_hbm.at[:, pl.ds(0, tile_v)], sems.at[s]
                ).wait()

    return body


def _tail_body(e_ref, w_ref, b_ref, o_ref):
    o_ref[...] = lax.dot_general(
        e_ref[...], w_ref[...],
        dimension_numbers=(((1,), (1,)), ((), ())),
        preferred_element_type=jnp.float32,
    ) + b_ref[...]


def _projection(embeds, linear_w, linear_b, tile_v=_TILE_V, nbuf=_NBUF):
    B, D = embeds.shape
    V = linear_w.shape[0]
    n_tiles = pl.cdiv(V, tile_v)
    n_full = V // tile_v
    bias2d = linear_b.reshape(1, V)

    main = pl.pallas_call(
        _make_main_body(n_full, tile_v, nbuf),
        grid=(n_full,),
        in_specs=[
            pl.BlockSpec((B, D), lambda j: (0, 0)),
            pl.BlockSpec((tile_v, D), lambda j: (j, 0)),
            pl.BlockSpec((1, tile_v), lambda j: (0, j)),
        ],
        out_specs=pl.BlockSpec(memory_space=pl.ANY),
        out_shape=jax.ShapeDtypeStruct((B, V), jnp.float32),
        scratch_shapes=[
            pltpu.VMEM((nbuf, B, tile_v), jnp.float32),
            pltpu.SemaphoreType.DMA((nbuf,)),
        ],
    )(embeds, linear_w, bias2d)

    if n_tiles == n_full:
        return main

    tail_v = V - n_full * tile_v
    tail = pl.pallas_call(
        _tail_body,
        grid=(1,),
        in_specs=[
            pl.BlockSpec((B, D), lambda j: (0, 0)),
            pl.BlockSpec((tail_v, D), lambda j: (0, 0)),
            pl.BlockSpec((1, tail_v), lambda j: (0, 0)),
        ],
        out_specs=pl.BlockSpec((B, tail_v), lambda j: (0, 0)),
        out_shape=jax.ShapeDtypeStruct((B, tail_v), jnp.float32),
    )(embeds, lax.slice(linear_w, (n_full * tile_v, 0), (V, D)),
      lax.slice(bias2d, (0, n_full * tile_v), (1, V)))
    return lax.dynamic_update_slice(main, tail, (0, n_full * tile_v))


def kernel(inputs, embedding_table, linear_w, linear_b):
    idx = inputs.astype(jnp.int32)
    embeds = _sc_gather(embedding_table, idx)
    return _projection(embeds, linear_w, linear_b)
